# baseline (speedup 1.0000x reference)
import functools
import math

import jax
import jax.numpy as jnp
from jax import lax
from jax.experimental import pallas as pl
from jax.experimental.pallas import tpu as pltpu
from jax.experimental.pallas import tpu_sc as plsc

_N_EV = 1048576
_N_TAB = 100000
_N_TABW = _N_TAB // 2
_TAB_SPLIT = 5
_LANES = 16
_NC = 2
_NS = 16
_NW = _NC * _NS
_EV_PER_TILE = _N_EV // _NW
_CHUNK = 2048
_NCHUNKS = _EV_PER_TILE // _CHUNK
_STEP = 4 * _LANES
_NSTEPS = _CHUNK // _STEP

_XCLIP = -math.log(1e-6)

_C7 = (0.010009289719164371, -0.05243753641843796, 0.1308334320783615,
       -0.2231658697128296, 0.3272257149219513, -0.4992850422859192,
       0.999967098236084, 2.554673130816809e-07)


def _softplus(x):
    u = jnp.exp(-jnp.abs(x))
    acc = _C7[0] * u + _C7[1]
    for c in _C7[2:]:
        acc = acc * u + c
    return jnp.maximum(x, 0.0) + acc


def _gather_bf16(tab, idx):
    word = plsc.load_gather(tab, [lax.shift_right_logical(idx, 1)])
    sh = (1 - (idx & 1)) * 16
    bits = (word << sh) & jnp.int32(-65536)
    return plsc.bitcast(bits, jnp.float32)


def _sc_body(t1h, t2h, wh, e1h, e2h, bih, pih, outh,
             tab1, tab2, bi0, bi1, pi0, pi1, e10, e11, e20, e21, wv, accv,
             s0, s1, st):
    cid = lax.axis_index("c")
    sid = lax.axis_index("s")
    wid = cid * _NS + sid
    ev0 = wid * _EV_PER_TILE

    bis = (bi0, bi1)
    pis = (pi0, pi1)
    e1s = (e10, e11)
    e2s = (e20, e21)
    sems = (s0, s1)

    tab_cps = []
    part = _N_TABW // _TAB_SPLIT
    for th, tv in ((t1h, tab1), (t2h, tab2)):
        for k in range(_TAB_SPLIT):
            sl = pl.ds(k * part, part)
            tab_cps.append(pltpu.async_copy(th.at[sl], tv.at[sl], st))

    def start_ev(ci):
        k = ci % 2
        base = ev0 + ci * _CHUNK
        sl = pl.ds(base, _CHUNK)
        return tuple(pltpu.async_copy(h.at[sl], bufs[k], sems[k])
                     for h, bufs in ((bih, bis), (pih, pis),
                                     (e1h, e1s), (e2h, e2s)))

    pend = start_ev(0)

    pltpu.sync_copy(wh, wv)
    w = wv[...]
    omw = 1.0 - w

    for cp in tab_cps:
        cp.wait()

    accs = tuple(jnp.zeros((_LANES,), jnp.float32) for _ in range(4))
    for ci in range(_NCHUNKS):
        k = ci % 2
        biv, piv, e1v, e2v = bis[k], pis[k], e1s[k], e2s[k]
        cur = pend
        if ci + 1 < _NCHUNKS:
            pend = start_ev(ci + 1)
        for c in cur:
            c.wait()

        def step(r, acc4, _refs=(biv, piv, e1v, e2v)):
            _biv, _piv, _e1v, _e2v = _refs
            off = r * _STEP
            out = []
            for q in range(4):
                sl = pl.ds(off + q * _LANES, _LANES)
                b = _gather_bf16(tab1, _biv[sl])
                p = _gather_bf16(tab2, _piv[sl])
                x = w * b + omw * p
                x = jnp.minimum(jnp.maximum(x, -_XCLIP), _XCLIP)
                sp = _softplus(x)
                e1 = _e1v[sl]
                out.append(acc4[q] + (e1 * x - (e1 + _e2v[sl]) * sp))
            return tuple(out)

        accs = lax.fori_loop(0, _NSTEPS, step, accs)

    accv[...] = (accs[0] + accs[1]) + (accs[2] + accs[3])
    pltpu.sync_copy(accv, outh.at[wid])


_sc_call = functools.partial(
    pl.kernel,
    out_type=jax.ShapeDtypeStruct((_NW, _LANES), jnp.float32),
    mesh=plsc.VectorSubcoreMesh(core_axis_name="c", subcore_axis_name="s"),
    compiler_params=pltpu.CompilerParams(needs_layout_passes=False),
    scratch_types=[
        pltpu.VMEM((_N_TABW,), jnp.int32),
        pltpu.VMEM((_N_TABW,), jnp.int32),
        pltpu.VMEM((_CHUNK,), jnp.int32),
        pltpu.VMEM((_CHUNK,), jnp.int32),
        pltpu.VMEM((_CHUNK,), jnp.int32),
        pltpu.VMEM((_CHUNK,), jnp.int32),
        pltpu.VMEM((_CHUNK,), jnp.float32),
        pltpu.VMEM((_CHUNK,), jnp.float32),
        pltpu.VMEM((_CHUNK,), jnp.float32),
        pltpu.VMEM((_CHUNK,), jnp.float32),
        pltpu.VMEM((_LANES,), jnp.float32),
        pltpu.VMEM((_LANES,), jnp.float32),
        pltpu.SemaphoreType.DMA,
        pltpu.SemaphoreType.DMA,
        pltpu.SemaphoreType.DMA,
    ],
)(_sc_body)


def kernel(v1, v2, weight, event1, event2, batter_idx, pitcher_idx):
    tb1 = lax.bitcast_convert_type(
        v1.astype(jnp.bfloat16).reshape(_N_TABW, 2), jnp.int32)
    tb2 = lax.bitcast_convert_type(
        v2.astype(jnp.bfloat16).reshape(_N_TABW, 2), jnp.int32)
    w16 = jnp.broadcast_to(weight.astype(jnp.float32), (_LANES,))
    parts = _sc_call(tb1, tb2, w16, event1, event2, batter_idx, pitcher_idx)
    return jnp.sum(parts)


# speedup vs baseline: 1.0004x; 1.0004x over previous
import functools
import math

import jax
import jax.numpy as jnp
from jax import lax
from jax.experimental import pallas as pl
from jax.experimental.pallas import tpu as pltpu
from jax.experimental.pallas import tpu_sc as plsc

_N_EV = 1048576
_N_TAB = 100000
_N_TABW = _N_TAB // 2
_TAB_SPLIT = 5
_LANES = 16
_NC = 2
_NS = 16
_NW = _NC * _NS
_EV_PER_TILE = _N_EV // _NW
_CHUNK = 2048
_NCHUNKS = _EV_PER_TILE // _CHUNK
_STEP = 2 * _LANES
_NSTEPS = _CHUNK // _STEP

_XCLIP = -math.log(1e-6)

_C7 = (0.010009289719164371, -0.05243753641843796, 0.1308334320783615,
       -0.2231658697128296, 0.3272257149219513, -0.4992850422859192,
       0.999967098236084, 2.554673130816809e-07)


def _softplus(x):
    u = jnp.exp(-jnp.abs(x))
    acc = _C7[0] * u + _C7[1]
    for c in _C7[2:]:
        acc = acc * u + c
    return jnp.maximum(x, 0.0) + acc


def _gather_bf16(tab, idx):
    word = plsc.bitcast(plsc.load_gather(tab, [lax.shift_right_logical(idx, 1)]), jnp.int32)
    sh = (1 - (idx & 1)) * 16
    bits = (word << sh) & jnp.int32(-65536)
    return plsc.bitcast(bits, jnp.float32)


def _sc_body(t1h, t2h, wh, e1h, e2h, bih, pih, outh,
             tab1, tab2, bi0, bi1, pi0, pi1, e10, e11, e20, e21, wv, accv,
             s0, s1, st):
    cid = lax.axis_index("c")
    sid = lax.axis_index("s")
    wid = cid * _NS + sid
    ev0 = wid * _EV_PER_TILE

    bis = (bi0, bi1)
    pis = (pi0, pi1)
    e1s = (e10, e11)
    e2s = (e20, e21)
    sems = (s0, s1)

    tab_cps = []
    part = _N_TABW // _TAB_SPLIT
    for th, tv in ((t1h, tab1), (t2h, tab2)):
        for k in range(_TAB_SPLIT):
            sl = pl.ds(k * part, part)
            tab_cps.append(pltpu.async_copy(th.at[sl], tv.at[sl], st))

    def start_ev(ci):
        k = ci % 2
        base = ev0 + ci * _CHUNK
        sl = pl.ds(base, _CHUNK)
        return tuple(pltpu.async_copy(h.at[sl], bufs[k], sems[k])
                     for h, bufs in ((bih, bis), (pih, pis),
                                     (e1h, e1s), (e2h, e2s)))

    pend = start_ev(0)

    pltpu.sync_copy(wh, wv)
    w = wv[...]
    omw = 1.0 - w

    for cp in tab_cps:
        cp.wait()

    accs = tuple(jnp.zeros((_LANES,), jnp.float32) for _ in range(2))
    for ci in range(_NCHUNKS):
        k = ci % 2
        biv, piv, e1v, e2v = bis[k], pis[k], e1s[k], e2s[k]
        cur = pend
        if ci + 1 < _NCHUNKS:
            pend = start_ev(ci + 1)
        for c in cur:
            c.wait()

        def step(r, acc4, _refs=(biv, piv, e1v, e2v)):
            _biv, _piv, _e1v, _e2v = _refs
            off = r * _STEP
            out = []
            for q in range(2):
                sl = pl.ds(off + q * _LANES, _LANES)
                b = _gather_bf16(tab1, _biv[sl])
                p = _gather_bf16(tab2, _piv[sl])
                x = w * b + omw * p
                x = jnp.minimum(jnp.maximum(x, -_XCLIP), _XCLIP)
                sp = _softplus(x)
                e1 = _e1v[sl]
                out.append(acc4[q] + (e1 * x - (e1 + _e2v[sl]) * sp))
            return tuple(out)

        accs = lax.fori_loop(0, _NSTEPS, step, accs)

    accv[...] = accs[0] + accs[1]
    pltpu.sync_copy(accv, outh.at[wid])


_sc_call = functools.partial(
    pl.kernel,
    out_type=jax.ShapeDtypeStruct((_NW, _LANES), jnp.float32),
    mesh=plsc.VectorSubcoreMesh(core_axis_name="c", subcore_axis_name="s"),
    compiler_params=pltpu.CompilerParams(needs_layout_passes=False),
    scratch_types=[
        pltpu.VMEM((_N_TABW,), jnp.float32),
        pltpu.VMEM((_N_TABW,), jnp.float32),
        pltpu.VMEM((_CHUNK,), jnp.int32),
        pltpu.VMEM((_CHUNK,), jnp.int32),
        pltpu.VMEM((_CHUNK,), jnp.int32),
        pltpu.VMEM((_CHUNK,), jnp.int32),
        pltpu.VMEM((_CHUNK,), jnp.float32),
        pltpu.VMEM((_CHUNK,), jnp.float32),
        pltpu.VMEM((_CHUNK,), jnp.float32),
        pltpu.VMEM((_CHUNK,), jnp.float32),
        pltpu.VMEM((_LANES,), jnp.float32),
        pltpu.VMEM((_LANES,), jnp.float32),
        pltpu.SemaphoreType.DMA,
        pltpu.SemaphoreType.DMA,
        pltpu.SemaphoreType.DMA,
    ],
)(_sc_body)


def kernel(v1, v2, weight, event1, event2, batter_idx, pitcher_idx):
    tb1 = lax.bitcast_convert_type(
        v1.astype(jnp.bfloat16).reshape(_N_TABW, 2), jnp.float32)
    tb2 = lax.bitcast_convert_type(
        v2.astype(jnp.bfloat16).reshape(_N_TABW, 2), jnp.float32)
    w16 = jnp.broadcast_to(weight.astype(jnp.float32), (_LANES,))
    parts = _sc_call(tb1, tb2, w16, event1, event2, batter_idx, pitcher_idx)
    return jnp.sum(parts)
